# Initial kernel scaffold; baseline (speedup 1.0000x reference)
import jax
import jax.numpy as jnp
from jax.experimental import pallas as pl


def kernel(descriptors, pts_src, pts_dst, invis_idx, height, width):
    raise NotImplementedError("write your pallas kernel here")


# speedup vs baseline: 40.9848x; 40.9848x over previous
import functools

import jax
import jax.numpy as jnp
from jax.experimental import pallas as pl
from jax.experimental.pallas import tpu as pltpu

_RADIUS = 4.0
_B = 8
_N = 512
_D = 256


def _pair_body(scale_ref, ps_ref, pd_ref, dst_ref, srct_ref, out_ref, acc_ref):
    b = pl.program_id(0)
    fx = scale_ref[0]
    fy = scale_ref[1]

    pdx = (pd_ref[0, :, 0:1] + 1.0) * fx
    pdy = (pd_ref[0, :, 1:2] + 1.0) * fy
    psx = (ps_ref[0, 0:1, :] + 1.0) * fx
    psy = (ps_ref[0, 1:2, :] + 1.0) * fy
    dx = psx - pdx
    dy = psy - pdy
    dist = jnp.sqrt(dx * dx + dy * dy)
    rowio = jax.lax.broadcasted_iota(jnp.int32, (_N, _N), 0)
    colio = jax.lax.broadcasted_iota(jnp.int32, (_N, _N), 1)
    match = (dist <= _RADIUS) & (rowio > colio)

    dst = dst_ref[0]
    srct = srct_ref[0]
    num = jax.lax.dot_general(
        dst, srct, (((1,), (0,)), ((), ())),
        preferred_element_type=jnp.float32,
        precision=jax.lax.Precision.HIGHEST,
    )
    nd = jnp.sqrt(jnp.sum(dst * dst, axis=1)).reshape(_N, 1)
    ns = jnp.sqrt(jnp.sum(srct * srct, axis=0)).reshape(1, _N)
    cos = num / jnp.maximum(nd * ns, 1e-8)

    @pl.when(b == 0)
    def _():
        acc_ref[0, 0] = 0.0
        acc_ref[0, 1] = 0.0

    acc_ref[0, 0] += jnp.sum(jnp.where(match, 1.0 - cos, 0.0))
    acc_ref[0, 1] += jnp.sum(match.astype(jnp.float32))

    @pl.when(b == pl.num_programs(0) - 1)
    def _():
        out_ref[0, 0] = acc_ref[0, 0] / acc_ref[0, 1]


def kernel(descriptors, pts_src, pts_dst, invis_idx, height, width):
    del invis_idx
    scale = jnp.stack([
        (width - 1.0) * 0.5, (height - 1.0) * 0.5]).astype(jnp.float32)
    ps_t = pts_src.transpose(0, 2, 1)
    pd = pts_dst.reshape(_B * _B, _N, 2)
    desc_t = descriptors.transpose(0, 2, 1)

    out = pl.pallas_call(
        _pair_body,
        grid=(_B * _B,),
        in_specs=[
            pl.BlockSpec(memory_space=pltpu.SMEM),
            pl.BlockSpec((1, 2, _N), lambda b: (b % _B, 0, 0)),
            pl.BlockSpec((1, _N, 2), lambda b: (b, 0, 0)),
            pl.BlockSpec((1, _N, _D), lambda b: (b // _B, 0, 0)),
            pl.BlockSpec((1, _D, _N), lambda b: (b % _B, 0, 0)),
        ],
        out_specs=pl.BlockSpec(memory_space=pltpu.SMEM),
        out_shape=jax.ShapeDtypeStruct((1, 1), jnp.float32),
        scratch_shapes=[pltpu.SMEM((1, 2), jnp.float32)],
    )(scale, ps_t, pd, descriptors, desc_t)
    return out.reshape(())


# speedup vs baseline: 59.0167x; 1.4400x over previous
import functools

import jax
import jax.numpy as jnp
from jax.experimental import pallas as pl
from jax.experimental.pallas import tpu as pltpu

_RADIUS = 4.0
_B = 8
_N = 512
_D = 256


def _pair_body(scale_ref, ps_ref, pd_ref, dst_ref, srct_ref, out_ref, acc_ref):
    b = pl.program_id(0)
    fx = scale_ref[0]
    fy = scale_ref[1]

    pdx = (pd_ref[0, :, 0:1] + 1.0) * fx
    pdy = (pd_ref[0, :, 1:2] + 1.0) * fy
    psx = (ps_ref[0, 0:1, :] + 1.0) * fx
    psy = (ps_ref[0, 1:2, :] + 1.0) * fy
    dx = psx - pdx
    dy = psy - pdy
    dist = jnp.sqrt(dx * dx + dy * dy)
    rowio = jax.lax.broadcasted_iota(jnp.int32, (_N, _N), 0)
    colio = jax.lax.broadcasted_iota(jnp.int32, (_N, _N), 1)
    match = (dist <= _RADIUS) & (rowio > colio)

    dst = dst_ref[0]
    srct = srct_ref[0]
    num = jax.lax.dot_general(
        dst, srct, (((1,), (0,)), ((), ())),
        preferred_element_type=jnp.float32,
        precision=jax.lax.Precision.DEFAULT,
    )
    nd = jnp.sqrt(jnp.sum(dst * dst, axis=1)).reshape(_N, 1)
    ns = jnp.sqrt(jnp.sum(srct * srct, axis=0)).reshape(1, _N)
    cos = num / jnp.maximum(nd * ns, 1e-8)

    @pl.when(b == 0)
    def _():
        acc_ref[0, 0] = 0.0
        acc_ref[0, 1] = 0.0

    acc_ref[0, 0] += jnp.sum(jnp.where(match, 1.0 - cos, 0.0))
    acc_ref[0, 1] += jnp.sum(match.astype(jnp.float32))

    @pl.when(b == pl.num_programs(0) - 1)
    def _():
        out_ref[0, 0] = acc_ref[0, 0] / acc_ref[0, 1]


def kernel(descriptors, pts_src, pts_dst, invis_idx, height, width):
    del invis_idx
    scale = jnp.stack([
        (width - 1.0) * 0.5, (height - 1.0) * 0.5]).astype(jnp.float32)
    ps_t = pts_src.transpose(0, 2, 1)
    pd = pts_dst.reshape(_B * _B, _N, 2)
    desc_t = descriptors.transpose(0, 2, 1)

    out = pl.pallas_call(
        _pair_body,
        grid=(_B * _B,),
        in_specs=[
            pl.BlockSpec(memory_space=pltpu.SMEM),
            pl.BlockSpec((1, 2, _N), lambda b: (b % _B, 0, 0)),
            pl.BlockSpec((1, _N, 2), lambda b: (b, 0, 0)),
            pl.BlockSpec((1, _N, _D), lambda b: (b // _B, 0, 0)),
            pl.BlockSpec((1, _D, _N), lambda b: (b % _B, 0, 0)),
        ],
        out_specs=pl.BlockSpec(memory_space=pltpu.SMEM),
        out_shape=jax.ShapeDtypeStruct((1, 1), jnp.float32),
        scratch_shapes=[pltpu.SMEM((1, 2), jnp.float32)],
    )(scale, ps_t, pd, descriptors, desc_t)
    return out.reshape(())
